# baseline (speedup 1.0000x reference)
import functools

import jax
import jax.numpy as jnp
from jax import lax
from jax.experimental import pallas as pl
from jax.experimental.pallas import tpu as pltpu
from jax.experimental.pallas import tpu_sc as plsc

VOCAB = 1_000_000
D = 64
B = 4096
S = 200
EPD = 2
T = S * EPD
NC, NS = 2, 16
NW = NC * NS
EPW = B // NW
RPW = EPW // EPD
NLAB = 10
UNROLL = 4


def _sc_pool(table, x2):
    mesh = plsc.VectorSubcoreMesh(core_axis_name="c", subcore_axis_name="s")

    @functools.partial(
        pl.kernel,
        out_type=jax.ShapeDtypeStruct((B, D), jnp.float32),
        mesh=mesh,
        compiler_params=pltpu.CompilerParams(use_tc_tiling_on_sc=False),
        scratch_types=[
            pltpu.VMEM((RPW, T), jnp.int32),
            pltpu.VMEM((2, T, D), jnp.float32),
            pltpu.VMEM((EPW, D), jnp.float32),
            pltpu.SemaphoreType.DMA((2,)),
        ],
    )
    def k(table_hbm, x_hbm, out_hbm, idx_v, bufs, bow_v, sems):
        wid = lax.axis_index("s") * NC + lax.axis_index("c")
        row0 = wid * RPW
        pltpu.sync_copy(x_hbm.at[pl.ds(row0, RPW)], idx_v)

        def fire(r, slot):
            pltpu.async_copy(
                table_hbm.at[idx_v.at[r]], bufs.at[slot], sems.at[slot]
            )

        def wait(r, slot):
            pltpu.make_async_copy(
                table_hbm.at[idx_v.at[r]], bufs.at[slot], sems.at[slot]
            ).wait()

        fire(0, 0)
        fire(1, 1)

        scale = jnp.float32(1.0 / S)

        def rloop(i, _):
            for p in range(2):
                r = i * 2 + p
                wait(r, p)
                for ei in range(EPD):
                    acc = (jnp.zeros((16,), jnp.float32),) * 8

                    def tbody(t, a, _p=p, _base=ei * S):
                        new = list(a)
                        base = _base + t * UNROLL
                        for u in range(UNROLL):
                            for j in range(4):
                                new[(u % 2) * 4 + j] = (
                                    new[(u % 2) * 4 + j]
                                    + bufs[_p, base + u, pl.ds(16 * j, 16)]
                                )
                        return tuple(new)

                    acc = lax.fori_loop(0, S // UNROLL, tbody, acc)
                    e = r * EPD + ei
                    for j in range(4):
                        bow_v[e, pl.ds(16 * j, 16)] = (
                            acc[j] + acc[4 + j]
                        ) * scale
                fire(jnp.minimum(r + 2, RPW - 1), p)
            return 0

        lax.fori_loop(0, RPW // 2, rloop, 0)

        for p in range(2):
            wait(RPW - 1, p)

        pltpu.sync_copy(bow_v, out_hbm.at[pl.ds(wid * EPW, EPW)])

    return k(table, x2)


VC = 10_240


def _tc_detile(table_t):
    def body(in_ref, eye_ref, out_ref):
        out_ref[...] = lax.dot_general(
            in_ref[...],
            eye_ref[...],
            (((0,), (0,)), ((), ())),
            preferred_element_type=jnp.float32,
        )

    eye = jnp.eye(D, dtype=jnp.float32)
    return pl.pallas_call(
        body,
        grid=((VOCAB + VC - 1) // VC,),
        in_specs=[
            pl.BlockSpec((D, VC), lambda i: (0, i)),
            pl.BlockSpec((D, D), lambda i: (0, 0)),
        ],
        out_specs=pl.BlockSpec((VC, D), lambda i: (i, 0)),
        out_shape=jax.ShapeDtypeStruct((VOCAB, D), jnp.float32),
    )(table_t, eye)


def _tc_head(bow, W, b):
    def body(bow_ref, w_ref, b_ref, out_ref):
        logits = (
            jnp.dot(bow_ref[...], w_ref[...], preferred_element_type=jnp.float32)
            + b_ref[...]
        )
        m = jnp.max(logits, axis=1, keepdims=True)
        s = logits - m
        lse = jnp.log(jnp.sum(jnp.exp(s), axis=1, keepdims=True))
        out_ref[...] = s - lse

    return pl.pallas_call(
        body,
        out_shape=jax.ShapeDtypeStruct((B, NLAB), jnp.float32),
    )(bow, W, b.reshape(1, NLAB))


@jax.jit
def kernel(x, table, W, b):
    x2 = x.reshape(B // EPD, T).astype(jnp.int32)
    table_rm = _tc_detile(table.T)
    bow = _sc_pool(table_rm, x2)
    return _tc_head(bow, W, b)


# speedup vs baseline: 1.0173x; 1.0173x over previous
import functools

import jax
import jax.numpy as jnp
from jax import lax
from jax.experimental import pallas as pl
from jax.experimental.pallas import tpu as pltpu
from jax.experimental.pallas import tpu_sc as plsc

VOCAB = 1_000_000
D = 64
B = 4096
S = 200
EPD = 2
T = S * EPD
NC, NS = 2, 16
NW = NC * NS
EPW = B // NW
RPW = EPW // EPD
NLAB = 10
UNROLL = 4


def _sc_pool(table, x2):
    mesh = plsc.VectorSubcoreMesh(core_axis_name="c", subcore_axis_name="s")

    @functools.partial(
        pl.kernel,
        out_type=jax.ShapeDtypeStruct((B, D), jnp.float32),
        mesh=mesh,
        compiler_params=pltpu.CompilerParams(use_tc_tiling_on_sc=False),
        scratch_types=[
            pltpu.VMEM((RPW, T), jnp.int32),
            pltpu.VMEM((2, T, D), jnp.float32),
            pltpu.VMEM((EPW, D), jnp.float32),
            pltpu.SemaphoreType.DMA((2,)),
        ],
    )
    def k(table_hbm, x_hbm, out_hbm, idx_v, bufs, bow_v, sems):
        wid = lax.axis_index("s") * NC + lax.axis_index("c")
        row0 = wid * RPW
        pltpu.sync_copy(x_hbm.at[pl.ds(row0, RPW)], idx_v)

        def fire(r, slot):
            pltpu.async_copy(
                table_hbm.at[idx_v.at[r]], bufs.at[slot], sems.at[slot]
            )

        def wait(r, slot):
            pltpu.make_async_copy(
                table_hbm.at[idx_v.at[r]], bufs.at[slot], sems.at[slot]
            ).wait()

        fire(0, 0)
        fire(1, 1)

        scale = jnp.float32(1.0 / S)

        def rloop(i, _):
            for p in range(2):
                r = i * 2 + p
                wait(r, p)
                for ei in range(EPD):
                    acc = (jnp.zeros((16,), jnp.float32),) * 8

                    def tbody(t, a, _p=p, _base=ei * S):
                        new = list(a)
                        base = _base + t * UNROLL
                        for u in range(UNROLL):
                            for j in range(4):
                                new[(u % 2) * 4 + j] = (
                                    new[(u % 2) * 4 + j]
                                    + bufs[_p, base + u, pl.ds(16 * j, 16)]
                                )
                        return tuple(new)

                    acc = lax.fori_loop(0, S // UNROLL, tbody, acc)
                    e = r * EPD + ei
                    for j in range(4):
                        bow_v[e, pl.ds(16 * j, 16)] = (
                            acc[j] + acc[4 + j]
                        ) * scale
                fire(jnp.minimum(r + 2, RPW - 1), p)
            return 0

        lax.fori_loop(0, RPW // 2, rloop, 0)

        for p in range(2):
            wait(RPW - 1, p)

        pltpu.sync_copy(bow_v, out_hbm.at[pl.ds(wid * EPW, EPW)])

    return k(table, x2)


VC = 20_480


def _tc_detile(table_t):
    def body(in_ref, eye_ref, out_ref):
        out_ref[...] = lax.dot_general(
            in_ref[...],
            eye_ref[...],
            (((0,), (0,)), ((), ())),
            preferred_element_type=jnp.float32,
        )

    eye = jnp.eye(D, dtype=jnp.float32)
    return pl.pallas_call(
        body,
        grid=((VOCAB + VC - 1) // VC,),
        in_specs=[
            pl.BlockSpec((D, VC), lambda i: (0, i)),
            pl.BlockSpec((D, D), lambda i: (0, 0)),
        ],
        out_specs=pl.BlockSpec((VC, D), lambda i: (i, 0)),
        out_shape=jax.ShapeDtypeStruct((VOCAB, D), jnp.float32),
    )(table_t, eye)


def _tc_head(bow, W, b):
    def body(bow_ref, w_ref, b_ref, out_ref):
        logits = (
            jnp.dot(bow_ref[...], w_ref[...], preferred_element_type=jnp.float32)
            + b_ref[...]
        )
        m = jnp.max(logits, axis=1, keepdims=True)
        s = logits - m
        lse = jnp.log(jnp.sum(jnp.exp(s), axis=1, keepdims=True))
        out_ref[...] = s - lse

    return pl.pallas_call(
        body,
        out_shape=jax.ShapeDtypeStruct((B, NLAB), jnp.float32),
    )(bow, W, b.reshape(1, NLAB))


@jax.jit
def kernel(x, table, W, b):
    x2 = x.reshape(B // EPD, T).astype(jnp.int32)
    table_rm = _tc_detile(table.T)
    bow = _sc_pool(table_rm, x2)
    return _tc_head(bow, W, b)


# speedup vs baseline: 2.2085x; 2.1708x over previous
import functools

import jax
import jax.numpy as jnp
from jax import lax
from jax.experimental import pallas as pl
from jax.experimental.pallas import tpu as pltpu
from jax.experimental.pallas import tpu_sc as plsc

VOCAB = 1_000_000
D = 64
B = 4096
S = 200
EPD = 2
T = S * EPD
NC, NS = 2, 16
NW = NC * NS
EPW = B // NW
RPW = EPW // EPD
NLAB = 10
UNROLL = 4

VC2 = 16_384
NBLK = 31
VROWS = 2 * NBLK * VC2


def _tc_detile(table_t):
    def body(in_a, in_b, eye_ref, out_ref):
        out_ref[:, 0:D] = lax.dot_general(
            in_a[...], eye_ref[...], (((0,), (0,)), ((), ())),
            preferred_element_type=jnp.float32,
        )
        out_ref[:, D : 2 * D] = lax.dot_general(
            in_b[...], eye_ref[...], (((0,), (0,)), ((), ())),
            preferred_element_type=jnp.float32,
        )

    eye = jnp.eye(D, dtype=jnp.float32)
    return pl.pallas_call(
        body,
        grid=(NBLK,),
        in_specs=[
            pl.BlockSpec((D, VC2), lambda i: (0, 2 * i)),
            pl.BlockSpec((D, VC2), lambda i: (0, 2 * i + 1)),
            pl.BlockSpec((D, D), lambda i: (0, 0)),
        ],
        out_specs=pl.BlockSpec((VC2, 2 * D), lambda i: (i, 0)),
        out_shape=jax.ShapeDtypeStruct((NBLK * VC2, 2 * D), jnp.float32),
    )(table_t, table_t, eye)


def _sc_pool(table, x2):
    mesh = plsc.VectorSubcoreMesh(core_axis_name="c", subcore_axis_name="s")

    @functools.partial(
        pl.kernel,
        out_type=jax.ShapeDtypeStruct((B, D), jnp.float32),
        mesh=mesh,
        compiler_params=pltpu.CompilerParams(use_tc_tiling_on_sc=False),
        scratch_types=[
            pltpu.VMEM((RPW, T), jnp.int32),
            pltpu.VMEM((2, T, D), jnp.float32),
            pltpu.VMEM((EPW, D), jnp.float32),
            pltpu.SemaphoreType.DMA((2,)),
        ],
    )
    def k(table_hbm, x_hbm, out_hbm, idx_v, bufs, bow_v, sems):
        wid = lax.axis_index("s") * NC + lax.axis_index("c")
        row0 = wid * RPW
        pltpu.sync_copy(x_hbm.at[pl.ds(row0, RPW)], idx_v)

        hi_mask = jnp.int32(~(2 * VC2 - 1))
        lo_mask = jnp.int32(VC2 - 1)

        def remap(i, _):
            row = i // (T // 16)
            col = 16 * (i % (T // 16))
            t = idx_v[row, pl.ds(col, 16)]
            r = (
                (t & hi_mask)
                + 2 * (t & lo_mask)
                + ((t >> 14) & jnp.int32(1))
            )
            idx_v[row, pl.ds(col, 16)] = r
            return 0

        lax.fori_loop(0, RPW * (T // 16), remap, 0)

        def fire(r, slot):
            pltpu.async_copy(
                table_hbm.at[idx_v.at[r]], bufs.at[slot], sems.at[slot]
            )

        def wait(r, slot):
            pltpu.make_async_copy(
                table_hbm.at[idx_v.at[r]], bufs.at[slot], sems.at[slot]
            ).wait()

        fire(0, 0)
        fire(1, 1)

        scale = jnp.float32(1.0 / S)

        def rloop(i, _):
            for p in range(2):
                r = i * 2 + p
                wait(r, p)
                for ei in range(EPD):
                    acc = (jnp.zeros((16,), jnp.float32),) * 8

                    def tbody(t, a, _p=p, _base=ei * S):
                        new = list(a)
                        base = _base + t * UNROLL
                        for u in range(UNROLL):
                            for j in range(4):
                                new[(u % 2) * 4 + j] = (
                                    new[(u % 2) * 4 + j]
                                    + bufs[_p, base + u, pl.ds(16 * j, 16)]
                                )
                        return tuple(new)

                    acc = lax.fori_loop(0, S // UNROLL, tbody, acc)
                    e = r * EPD + ei
                    for j in range(4):
                        bow_v[e, pl.ds(16 * j, 16)] = (
                            acc[j] + acc[4 + j]
                        ) * scale
                fire(jnp.minimum(r + 2, RPW - 1), p)
            return 0

        lax.fori_loop(0, RPW // 2, rloop, 0)

        for p in range(2):
            wait(RPW - 1, p)

        pltpu.sync_copy(bow_v, out_hbm.at[pl.ds(wid * EPW, EPW)])

    return k(table, x2)


def _tc_head(bow, W, b):
    def body(bow_ref, w_ref, b_ref, out_ref):
        logits = (
            jnp.dot(bow_ref[...], w_ref[...], preferred_element_type=jnp.float32)
            + b_ref[...]
        )
        m = jnp.max(logits, axis=1, keepdims=True)
        s = logits - m
        lse = jnp.log(jnp.sum(jnp.exp(s), axis=1, keepdims=True))
        out_ref[...] = s - lse

    return pl.pallas_call(
        body,
        out_shape=jax.ShapeDtypeStruct((B, NLAB), jnp.float32),
    )(bow, W, b.reshape(1, NLAB))


@jax.jit
def kernel(x, table, W, b):
    x2 = x.reshape(B // EPD, T).astype(jnp.int32)
    table_rm = _tc_detile(table.T).reshape(VROWS, D)
    bow = _sc_pool(table_rm, x2)
    return _tc_head(bow, W, b)
